# baseline (speedup 1.0000x reference)
import functools

import jax
import jax.numpy as jnp
from jax import lax
from jax.experimental import pallas as pl
from jax.experimental.pallas import tpu as pltpu
from jax.experimental.pallas import tpu_sc as plsc

N = 10000
NPAD = 10112
E = 160000
D = 256
DH = 128
DIMS = 64
G = 128
ATOM_F = 9
ATOM_V = 119
BOND_F = 3
BOND_V = 6
EPS = 1e-7
POWER = 4

NB = 1000
EBC = 1280
EPT = E // 16
CB = 80
NCHUNK = EPT // CB
STRIPE = NPAD // 16

_HI = lax.Precision.HIGHEST
_DF = None


def _ln(h, s, b):
    m = jnp.mean(h, axis=-1, keepdims=True)
    v = jnp.mean((h - m) ** 2, axis=-1, keepdims=True)
    return (h - m) / jnp.sqrt(v + 1e-5) * s + b


def _encode_body(x_ref, atom_ref, vnt_ref, hinit_ref, h0s_ref):
    x = x_ref[...]
    acc = jnp.zeros((NB, D), jnp.float32)
    for f in range(ATOM_F):
        col = x[:, f:f + 1]
        ids = lax.broadcasted_iota(jnp.int32, (NB, ATOM_V), 1)
        oh = jnp.where(col == ids, 1.0, 0.0).astype(jnp.float32)
        acc = acc + jax.lax.dot_general(
            oh, atom_ref[f], (((1,), (0,)), ((), ())),
            precision=_HI, preferred_element_type=jnp.float32)
    hinit_ref[...] = acc
    h0 = acc + vnt_ref[...]
    h0s_ref[0] = h0[:, :DH]
    h0s_ref[1] = h0[:, DH:]


def _encode(x, atom_emb, vn_table):
    return pl.pallas_call(
        _encode_body,
        grid=(N // NB,),
        in_specs=[
            pl.BlockSpec((NB, ATOM_F), lambda i: (i, 0)),
            pl.BlockSpec((ATOM_F, ATOM_V, D), lambda i: (0, 0, 0)),
            pl.BlockSpec((1, D), lambda i: (0, 0)),
        ],
        out_specs=[
            pl.BlockSpec((NB, D), lambda i: (i, 0)),
            pl.BlockSpec((2, NB, DH), lambda i: (0, i, 0)),
        ],
        out_shape=[
            jax.ShapeDtypeStruct((N, D), jnp.float32),
            jax.ShapeDtypeStruct((2, N, DH), jnp.float32),
        ],
    )(x, atom_emb, vn_table)


def _bond_body(bond_ref, out_ref):
    c = lax.broadcasted_iota(jnp.int32, (216, 1), 0)
    digits = [c // 36, (c // 6) % 6, c % 6]
    acc = jnp.zeros((216, D), jnp.float32)
    for f in range(BOND_F):
        ids = lax.broadcasted_iota(jnp.int32, (216, BOND_V), 1)
        oh = jnp.where(digits[f] == ids, 1.0, 0.0).astype(jnp.float32)
        acc = acc + jax.lax.dot_general(
            oh, bond_ref[f], (((1,), (0,)), ((), ())),
            precision=_HI, preferred_element_type=jnp.float32)
    out_ref[0] = acc[:, :DH]
    out_ref[1] = acc[:, DH:]


def _bond_table(bond_emb):
    return pl.pallas_call(
        _bond_body,
        in_specs=[pl.BlockSpec((BOND_F, BOND_V, D), lambda: (0, 0, 0))],
        out_specs=pl.BlockSpec((2, 216, DH), lambda: (0, 0, 0)),
        out_shape=jax.ShapeDtypeStruct((2, 216, DH), jnp.float32),
    )(bond_emb)


def _code_body(attr_ref, ei_ref, code_ref, srcpre_ref):
    a = attr_ref[...]
    code_ref[...] = a[0:1, :] * 36 + a[1:2, :] * 6 + a[2:3, :]
    src_row = ei_ref[0:1, :]
    srcpre_ref[0:1, :] = src_row
    srcpre_ref[1:2, :] = src_row + N


def _edge_codes(attr_t, edge_index):
    return pl.pallas_call(
        _code_body,
        grid=(E // EBC,),
        in_specs=[
            pl.BlockSpec((BOND_F, EBC), lambda e: (0, e)),
            pl.BlockSpec((2, EBC), lambda e: (0, e)),
        ],
        out_specs=[
            pl.BlockSpec((1, EBC), lambda e: (0, e)),
            pl.BlockSpec((2, EBC), lambda e: (0, e)),
        ],
        out_shape=[
            jax.ShapeDtypeStruct((1, E), jnp.int32),
            jax.ShapeDtypeStruct((2, E), jnp.int32),
        ],
    )(attr_t, edge_index)


NSTEP = 6


def _agg_sc_body(h_ref, srcpre_ref, dst_ref, code_ref, emb_ref, zero_ref,
                 m_ref, emb_v, src0_v, src1_v, src2_v, dst0_v, dst1_v,
                 dst2_v, code0_v, code1_v, code2_v, rows0_v, rows1_v,
                 macc, gsem0, gsem1, isem0, isem1, isem2):
    c = lax.axis_index("c")
    s = lax.axis_index("s")
    base = s * EPT
    ibufs = ((src0_v, dst0_v, code0_v, isem0),
             (src1_v, dst1_v, code1_v, isem1),
             (src2_v, dst2_v, code2_v, isem2))
    rbufs = ((rows0_v, gsem0), (rows1_v, gsem1))

    pltpu.sync_copy(emb_ref.at[pl.ds(c * 216, 216)], emb_v)
    pltpu.sync_copy(zero_ref.at[pl.ds(s * STRIPE, STRIPE)],
                    macc.at[pl.ds(s * STRIPE, STRIPE)])
    plsc.subcore_barrier()

    def stage_idx(j, ib):
        src_v, dst_v, code_v, isem = ibufs[ib]
        off = base + j * CB
        pltpu.make_async_copy(
            srcpre_ref.at[pl.ds(c * E + off, CB)], src_v, isem).start()
        pltpu.make_async_copy(dst_ref.at[pl.ds(off, CB)], dst_v,
                              isem).start()
        pltpu.make_async_copy(code_ref.at[pl.ds(off, CB)],
                              code_v.at[pl.ds(0, CB)], isem).start()

    def wait_idx(ib):
        src_v, dst_v, code_v, isem = ibufs[ib]
        pltpu.make_async_copy(dst_ref.at[pl.ds(0, CB)], src_v, isem).wait()
        pltpu.make_async_copy(dst_ref.at[pl.ds(0, CB)], dst_v, isem).wait()
        pltpu.make_async_copy(dst_ref.at[pl.ds(0, CB)],
                              code_v.at[pl.ds(0, CB)], isem).wait()

    def start_gather(ib, rb):
        src_v = ibufs[ib][0]
        rows_v, gsem = rbufs[rb]
        pltpu.make_async_copy(h_ref.at[src_v], rows_v, gsem).start()

    def step(j, ib, rb, stage_ok, gather_ok):
        if stage_ok:
            stage_idx(j + 2, (ib + 2) % 3)
        if gather_ok:
            wait_idx((ib + 1) % 3)
            start_gather((ib + 1) % 3, (rb + 1) % 2)
        rows_v, gsem = rbufs[rb]
        _, dst_v, code_v, _ = ibufs[ib]
        pltpu.make_async_copy(h_ref.at[ibufs[ib][0]], rows_v, gsem).wait()

        @plsc.parallel_loop(0, CB, unroll=8)
        def _edge(e):
            ce = code_v[pl.ds(e, 16)][0]
            for k in range(DH // 16):
                sl = pl.ds(k * 16, 16)
                v = rows_v[e, sl] + emb_v[ce, sl]
                rows_v[e, sl] = jnp.maximum(v, 0.0) + EPS

        pltpu.sync_copy(rows_v, macc.at[dst_v], add=True)

    stage_idx(0, 0)
    stage_idx(1, 1)
    wait_idx(0)
    start_gather(0, 0)

    ngroup = (NCHUNK - NSTEP + 1) // NSTEP

    def group_body(jj, carry):
        for b in range(NSTEP):
            step(jj * NSTEP + b, b % 3, b % 2,
                 stage_ok=True, gather_ok=True)
        return carry

    lax.fori_loop(0, ngroup, group_body, 0, unroll=False)
    for j in range(ngroup * NSTEP, NCHUNK):
        step(j, j % 3, j % 2,
             stage_ok=(j + 2 < NCHUNK), gather_ok=(j + 1 < NCHUNK))

    plsc.subcore_barrier()
    pltpu.sync_copy(macc.at[pl.ds(s * STRIPE, STRIPE)],
                    m_ref.at[pl.ds(c * NPAD + s * STRIPE, STRIPE)])


def _agg_sc(h_split2, srcpre, dst, code, emb432, zeros):
    mesh = plsc.VectorSubcoreMesh(core_axis_name="c", subcore_axis_name="s")
    kern = functools.partial(
        pl.kernel,
        mesh=mesh,
        out_type=jax.ShapeDtypeStruct((2 * NPAD, DH), jnp.float32),
        scratch_types=[
            pltpu.VMEM((216, DH), jnp.float32),
            pltpu.VMEM((CB,), jnp.int32),
            pltpu.VMEM((CB,), jnp.int32),
            pltpu.VMEM((CB,), jnp.int32),
            pltpu.VMEM((CB,), jnp.int32),
            pltpu.VMEM((CB,), jnp.int32),
            pltpu.VMEM((CB,), jnp.int32),
            pltpu.VMEM((CB + 16,), jnp.int32),
            pltpu.VMEM((CB + 16,), jnp.int32),
            pltpu.VMEM((CB + 16,), jnp.int32),
            pltpu.VMEM((CB, DH), jnp.float32),
            pltpu.VMEM((CB, DH), jnp.float32),
            pltpu.VMEM_SHARED((NPAD, DH), jnp.float32),
            pltpu.SemaphoreType.DMA,
            pltpu.SemaphoreType.DMA,
            pltpu.SemaphoreType.DMA,
            pltpu.SemaphoreType.DMA,
            pltpu.SemaphoreType.DMA,
        ],
    )(_agg_sc_body)
    return kern(h_split2, srcpre, dst, code, emb432, zeros)


def _layer_body(h2s_ref, m_ref, res_ref, W_ref, b_ref, fW_ref, fb_ref,
                hnew_ref, outl_ref):
    lo = h2s_ref[0] + m_ref[0]
    hi = h2s_ref[1] + m_ref[1]
    hn = jax.lax.dot_general(
        lo, W_ref[:DH, :], (((1,), (0,)), ((), ())),
        precision=_DF, preferred_element_type=jnp.float32)
    hn = hn + jax.lax.dot_general(
        hi, W_ref[DH:, :], (((1,), (0,)), ((), ())),
        precision=_DF, preferred_element_type=jnp.float32)
    hn = hn + b_ref[...] + res_ref[...]
    hnew_ref[...] = hn
    outl_ref[...] = jax.lax.dot_general(
        hn, fW_ref[...], (((1,), (0,)), ((), ())),
        precision=_DF, preferred_element_type=jnp.float32) + fb_ref[...]


def _layer_mm(h2s, m, res, W, b, fW, fb):
    return pl.pallas_call(
        _layer_body,
        grid=(N // NB,),
        in_specs=[
            pl.BlockSpec((2, NB, DH), lambda i: (0, i, 0)),
            pl.BlockSpec((2, NB, DH), lambda i: (0, i, 0)),
            pl.BlockSpec((NB, D), lambda i: (i, 0)),
            pl.BlockSpec((D, D), lambda i: (0, 0)),
            pl.BlockSpec((1, D), lambda i: (0, 0)),
            pl.BlockSpec((D, DIMS), lambda i: (0, 0)),
            pl.BlockSpec((1, DIMS), lambda i: (0, 0)),
        ],
        out_specs=[
            pl.BlockSpec((NB, D), lambda i: (i, 0)),
            pl.BlockSpec((NB, DIMS), lambda i: (i, 0)),
        ],
        out_shape=[
            jax.ShapeDtypeStruct((N, D), jnp.float32),
            jax.ShapeDtypeStruct((N, DIMS), jnp.float32),
        ],
    )(h2s, m, res, W, b, fW, fb)


def _vnsum_body(h_ref, s_ref, b_ref, batch_ref, h2_ref, vnsum_ref):
    i = pl.program_id(0)

    @pl.when(i == 0)
    def _():
        vnsum_ref[...] = jnp.zeros_like(vnsum_ref)

    h2 = jnp.maximum(_ln(h_ref[...], s_ref[...], b_ref[...]), 0.0)
    h2_ref[...] = h2
    gids = lax.broadcasted_iota(jnp.int32, (G, NB), 0)
    oh = jnp.where(gids == batch_ref[0], 1.0, 0.0).astype(jnp.float32)
    vnsum_ref[...] += jax.lax.dot_general(
        oh, h2, (((1,), (0,)), ((), ())),
        precision=_HI, preferred_element_type=jnp.float32)


def _vnsum(h, s, b, batch3d):
    return pl.pallas_call(
        _vnsum_body,
        grid=(N // NB,),
        in_specs=[
            pl.BlockSpec((NB, D), lambda i: (i, 0)),
            pl.BlockSpec((1, D), lambda i: (0, 0)),
            pl.BlockSpec((1, D), lambda i: (0, 0)),
            pl.BlockSpec((1, 1, NB), lambda i: (i, 0, 0)),
        ],
        out_specs=[
            pl.BlockSpec((NB, D), lambda i: (i, 0)),
            pl.BlockSpec((G, D), lambda i: (0, 0)),
        ],
        out_shape=[
            jax.ShapeDtypeStruct((N, D), jnp.float32),
            jax.ShapeDtypeStruct((G, D), jnp.float32),
        ],
    )(h, s, b, batch3d)


def _vnmlp_body(vnsum_ref, vnprev_ref, W1_ref, b1_ref, s_ref, b_ref,
                W2_ref, b2_ref, out_ref):
    vt = vnsum_ref[...] + vnprev_ref[...]
    t = jax.lax.dot_general(vt, W1_ref[...], (((1,), (0,)), ((), ())),
                            precision=_DF,
                            preferred_element_type=jnp.float32) + b1_ref[...]
    t = jnp.maximum(_ln(t, s_ref[...], b_ref[...]), 0.0)
    out_ref[...] = jax.lax.dot_general(
        t, W2_ref[...], (((1,), (0,)), ((), ())),
        precision=_DF, preferred_element_type=jnp.float32) + b2_ref[...]


def _vnmlp(vnsum, vnprev, W1, b1, s, b, W2, b2):
    full = lambda a, bb: pl.BlockSpec((a, bb), lambda: (0, 0))
    return pl.pallas_call(
        _vnmlp_body,
        in_specs=[full(G, D), full(G, D), full(D, D), full(1, D),
                  full(1, D), full(1, D), full(D, D), full(1, D)],
        out_specs=full(G, D),
        out_shape=jax.ShapeDtypeStruct((G, D), jnp.float32),
    )(vnsum, vnprev, W1, b1, s, b, W2, b2)


def _vnadd_body(h2_ref, vn_ref, batch_ref, out_ref):
    bcol = batch_ref[0].reshape(NB, 1)
    gids = lax.broadcasted_iota(jnp.int32, (NB, G), 1)
    oh = jnp.where(bcol == gids, 1.0, 0.0).astype(jnp.float32)
    v = h2_ref[...] + jax.lax.dot_general(
        oh, vn_ref[...], (((1,), (0,)), ((), ())),
        precision=_HI, preferred_element_type=jnp.float32)
    out_ref[0] = v[:, :DH]
    out_ref[1] = v[:, DH:]


def _vnadd(h2, vn, batch3d):
    return pl.pallas_call(
        _vnadd_body,
        grid=(N // NB,),
        in_specs=[
            pl.BlockSpec((NB, D), lambda i: (i, 0)),
            pl.BlockSpec((G, D), lambda i: (0, 0)),
            pl.BlockSpec((1, 1, NB), lambda i: (i, 0, 0)),
        ],
        out_specs=pl.BlockSpec((2, NB, DH), lambda i: (0, i, 0)),
        out_shape=jax.ShapeDtypeStruct((2, N, DH), jnp.float32),
    )(h2, vn, batch3d)


def kernel(x, edge_attr, edge_index, batch, atom_emb, bond_emb, vn_table,
           gcn_W, gcn_b, ln_scale, ln_bias, ffn_W, ffn_b,
           vn_W1, vn_b1, vn_ln_s, vn_ln_b, vn_W2, vn_b2):
    dst = edge_index[1]
    attr_t = edge_attr.T.reshape(BOND_F, E)
    batch3d = batch.reshape(N // NB, 1, NB)
    zeros_pad = jnp.zeros((NPAD, DH), jnp.float32)
    zeros_nd = jnp.zeros((N, D), jnp.float32)

    h_init, h0s = _encode(x, atom_emb, vn_table)
    emb432 = _bond_table(bond_emb).reshape(432, DH)
    code2d, srcpre2d = _edge_codes(attr_t, edge_index)
    code = code2d.reshape(E)
    srcpre = srcpre2d.reshape(2 * E)
    vn = jnp.broadcast_to(vn_table, (G, D))

    outs = []

    h2s = h0s
    m = _agg_sc(h2s.reshape(2 * N, DH), srcpre, dst, code, emb432,
                zeros_pad).reshape(2, NPAD, DH)
    h, out0 = _layer_mm(h2s, m, zeros_nd, gcn_W[0],
                        gcn_b[0].reshape(1, D), ffn_W[0],
                        ffn_b[0].reshape(1, DIMS))
    outs.append(out0)

    for l in range(1, POWER):
        h2, vnsum = _vnsum(h, ln_scale[l - 1].reshape(1, D),
                           ln_bias[l - 1].reshape(1, D), batch3d)
        vn = _vnmlp(vnsum, vn, vn_W1[l - 1], vn_b1[l - 1].reshape(1, D),
                    vn_ln_s[l - 1].reshape(1, D), vn_ln_b[l - 1].reshape(1, D),
                    vn_W2[l - 1], vn_b2[l - 1].reshape(1, D))
        h2s = _vnadd(h2, vn, batch3d)
        m = _agg_sc(h2s.reshape(2 * N, DH), srcpre, dst, code, emb432,
                    zeros_pad).reshape(2, NPAD, DH)
        h, out_l = _layer_mm(h2s, m, h, gcn_W[l],
                             gcn_b[l].reshape(1, D), ffn_W[l],
                             ffn_b[l].reshape(1, DIMS))
        outs.append(out_l)

    h_graph = jnp.concatenate(outs, axis=-1)
    return (h_graph, h_init)


# speedup vs baseline: 1.0106x; 1.0106x over previous
import functools

import jax
import jax.numpy as jnp
from jax import lax
from jax.experimental import pallas as pl
from jax.experimental.pallas import tpu as pltpu
from jax.experimental.pallas import tpu_sc as plsc

N = 10000
NPAD = 10112
E = 160000
D = 256
DH = 128
DIMS = 64
G = 128
ATOM_F = 9
ATOM_V = 119
BOND_F = 3
BOND_V = 6
EPS = 1e-7
POWER = 4

NB = 1000
EBC = 1280
EPT = E // 16
CB = 80
NCHUNK = EPT // CB
STRIPE = NPAD // 16

_HI = lax.Precision.HIGHEST
_DF = None


def _ln(h, s, b):
    m = jnp.mean(h, axis=-1, keepdims=True)
    v = jnp.mean((h - m) ** 2, axis=-1, keepdims=True)
    return (h - m) / jnp.sqrt(v + 1e-5) * s + b


def _encode_body(x_ref, atom_ref, vnt_ref, hinit_ref, h0s_ref):
    x = x_ref[...]
    acc = jnp.zeros((NB, D), jnp.float32)
    for f in range(ATOM_F):
        col = x[:, f:f + 1]
        ids = lax.broadcasted_iota(jnp.int32, (NB, ATOM_V), 1)
        oh = jnp.where(col == ids, 1.0, 0.0).astype(jnp.float32)
        acc = acc + jax.lax.dot_general(
            oh, atom_ref[f], (((1,), (0,)), ((), ())),
            precision=_HI, preferred_element_type=jnp.float32)
    hinit_ref[...] = acc
    h0 = acc + vnt_ref[...]
    h0s_ref[0] = h0[:, :DH]
    h0s_ref[1] = h0[:, DH:]


def _encode(x, atom_emb, vn_table):
    return pl.pallas_call(
        _encode_body,
        grid=(N // NB,),
        in_specs=[
            pl.BlockSpec((NB, ATOM_F), lambda i: (i, 0)),
            pl.BlockSpec((ATOM_F, ATOM_V, D), lambda i: (0, 0, 0)),
            pl.BlockSpec((1, D), lambda i: (0, 0)),
        ],
        out_specs=[
            pl.BlockSpec((NB, D), lambda i: (i, 0)),
            pl.BlockSpec((2, NB, DH), lambda i: (0, i, 0)),
        ],
        out_shape=[
            jax.ShapeDtypeStruct((N, D), jnp.float32),
            jax.ShapeDtypeStruct((2, N, DH), jnp.float32),
        ],
    )(x, atom_emb, vn_table)


def _bond_body(bond_ref, out_ref):
    c = lax.broadcasted_iota(jnp.int32, (216, 1), 0)
    digits = [c // 36, (c // 6) % 6, c % 6]
    acc = jnp.zeros((216, D), jnp.float32)
    for f in range(BOND_F):
        ids = lax.broadcasted_iota(jnp.int32, (216, BOND_V), 1)
        oh = jnp.where(digits[f] == ids, 1.0, 0.0).astype(jnp.float32)
        acc = acc + jax.lax.dot_general(
            oh, bond_ref[f], (((1,), (0,)), ((), ())),
            precision=_HI, preferred_element_type=jnp.float32)
    out_ref[0] = acc[:, :DH]
    out_ref[1] = acc[:, DH:]


def _bond_table(bond_emb):
    return pl.pallas_call(
        _bond_body,
        in_specs=[pl.BlockSpec((BOND_F, BOND_V, D), lambda: (0, 0, 0))],
        out_specs=pl.BlockSpec((2, 216, DH), lambda: (0, 0, 0)),
        out_shape=jax.ShapeDtypeStruct((2, 216, DH), jnp.float32),
    )(bond_emb)


def _code_body(attr_ref, ei_ref, code_ref, srcpre_ref):
    a = attr_ref[...]
    code_ref[...] = a[0:1, :] * 36 + a[1:2, :] * 6 + a[2:3, :]
    src_row = ei_ref[0:1, :]
    srcpre_ref[0:1, :] = src_row
    srcpre_ref[1:2, :] = src_row + N


def _edge_codes(attr_t, edge_index):
    return pl.pallas_call(
        _code_body,
        grid=(E // EBC,),
        in_specs=[
            pl.BlockSpec((BOND_F, EBC), lambda e: (0, e)),
            pl.BlockSpec((2, EBC), lambda e: (0, e)),
        ],
        out_specs=[
            pl.BlockSpec((1, EBC), lambda e: (0, e)),
            pl.BlockSpec((2, EBC), lambda e: (0, e)),
        ],
        out_shape=[
            jax.ShapeDtypeStruct((1, E), jnp.int32),
            jax.ShapeDtypeStruct((2, E), jnp.int32),
        ],
    )(attr_t, edge_index)


NSTEP = 6


def _agg_sc_body(h_ref, srcpre_ref, dst_ref, code_ref, emb_ref, zero_ref,
                 m_ref, emb_v, src0_v, src1_v, src2_v, dst0_v, dst1_v,
                 dst2_v, code0_v, code1_v, code2_v, rows0_v, rows1_v,
                 macc, gsem0, gsem1, isem0, isem1, isem2):
    c = lax.axis_index("c")
    s = lax.axis_index("s")
    base = s * EPT
    ibufs = ((src0_v, dst0_v, code0_v, isem0),
             (src1_v, dst1_v, code1_v, isem1),
             (src2_v, dst2_v, code2_v, isem2))
    rbufs = ((rows0_v, gsem0), (rows1_v, gsem1))

    pltpu.sync_copy(emb_ref.at[pl.ds(c * 216, 216)], emb_v)
    pltpu.sync_copy(zero_ref.at[pl.ds(s * STRIPE, STRIPE)],
                    macc.at[pl.ds(s * STRIPE, STRIPE)])
    plsc.subcore_barrier()

    def stage_idx(j, ib):
        src_v, dst_v, code_v, isem = ibufs[ib]
        off = base + j * CB
        pltpu.make_async_copy(
            srcpre_ref.at[pl.ds(c * E + off, CB)], src_v, isem).start()
        pltpu.make_async_copy(dst_ref.at[pl.ds(off, CB)], dst_v,
                              isem).start()
        pltpu.make_async_copy(code_ref.at[pl.ds(off, CB)],
                              code_v.at[pl.ds(0, CB)], isem).start()

    def wait_idx(ib):
        src_v, dst_v, code_v, isem = ibufs[ib]
        pltpu.make_async_copy(dst_ref.at[pl.ds(0, CB)], src_v, isem).wait()
        pltpu.make_async_copy(dst_ref.at[pl.ds(0, CB)], dst_v, isem).wait()
        pltpu.make_async_copy(dst_ref.at[pl.ds(0, CB)],
                              code_v.at[pl.ds(0, CB)], isem).wait()

    def start_gather(ib, rb):
        src_v = ibufs[ib][0]
        rows_v, gsem = rbufs[rb]
        pltpu.make_async_copy(h_ref.at[src_v], rows_v, gsem).start()

    def step(j, ib, rb, stage_ok, gather_ok):
        if stage_ok:
            stage_idx(j + 2, (ib + 2) % 3)
        if gather_ok:
            wait_idx((ib + 1) % 3)
            start_gather((ib + 1) % 3, (rb + 1) % 2)
        rows_v, gsem = rbufs[rb]
        _, dst_v, code_v, _ = ibufs[ib]
        pltpu.make_async_copy(h_ref.at[ibufs[ib][0]], rows_v, gsem).wait()

        @plsc.parallel_loop(0, CB, unroll=4)
        def _edge(e):
            ce = code_v[pl.ds(e, 16)][0]
            for k in range(DH // 16):
                sl = pl.ds(k * 16, 16)
                v = rows_v[e, sl] + emb_v[ce, sl]
                rows_v[e, sl] = jnp.maximum(v, 0.0) + EPS

        pltpu.sync_copy(rows_v, macc.at[dst_v], add=True)

    stage_idx(0, 0)
    stage_idx(1, 1)
    wait_idx(0)
    start_gather(0, 0)

    ngroup = (NCHUNK - NSTEP + 1) // NSTEP

    def group_body(jj, carry):
        for b in range(NSTEP):
            step(jj * NSTEP + b, b % 3, b % 2,
                 stage_ok=True, gather_ok=True)
        return carry

    lax.fori_loop(0, ngroup, group_body, 0, unroll=False)
    for j in range(ngroup * NSTEP, NCHUNK):
        step(j, j % 3, j % 2,
             stage_ok=(j + 2 < NCHUNK), gather_ok=(j + 1 < NCHUNK))

    plsc.subcore_barrier()
    pltpu.sync_copy(macc.at[pl.ds(s * STRIPE, STRIPE)],
                    m_ref.at[pl.ds(c * NPAD + s * STRIPE, STRIPE)])


def _agg_sc(h_split2, srcpre, dst, code, emb432, zeros):
    mesh = plsc.VectorSubcoreMesh(core_axis_name="c", subcore_axis_name="s")
    kern = functools.partial(
        pl.kernel,
        mesh=mesh,
        out_type=jax.ShapeDtypeStruct((2 * NPAD, DH), jnp.float32),
        scratch_types=[
            pltpu.VMEM((216, DH), jnp.float32),
            pltpu.VMEM((CB,), jnp.int32),
            pltpu.VMEM((CB,), jnp.int32),
            pltpu.VMEM((CB,), jnp.int32),
            pltpu.VMEM((CB,), jnp.int32),
            pltpu.VMEM((CB,), jnp.int32),
            pltpu.VMEM((CB,), jnp.int32),
            pltpu.VMEM((CB + 16,), jnp.int32),
            pltpu.VMEM((CB + 16,), jnp.int32),
            pltpu.VMEM((CB + 16,), jnp.int32),
            pltpu.VMEM((CB, DH), jnp.float32),
            pltpu.VMEM((CB, DH), jnp.float32),
            pltpu.VMEM_SHARED((NPAD, DH), jnp.float32),
            pltpu.SemaphoreType.DMA,
            pltpu.SemaphoreType.DMA,
            pltpu.SemaphoreType.DMA,
            pltpu.SemaphoreType.DMA,
            pltpu.SemaphoreType.DMA,
        ],
    )(_agg_sc_body)
    return kern(h_split2, srcpre, dst, code, emb432, zeros)


def _layer_body(h2s_ref, m_ref, res_ref, W_ref, b_ref, fW_ref, fb_ref,
                hnew_ref, outl_ref):
    lo = h2s_ref[0] + m_ref[0]
    hi = h2s_ref[1] + m_ref[1]
    hn = jax.lax.dot_general(
        lo, W_ref[:DH, :], (((1,), (0,)), ((), ())),
        precision=_DF, preferred_element_type=jnp.float32)
    hn = hn + jax.lax.dot_general(
        hi, W_ref[DH:, :], (((1,), (0,)), ((), ())),
        precision=_DF, preferred_element_type=jnp.float32)
    hn = hn + b_ref[...] + res_ref[...]
    hnew_ref[...] = hn
    outl_ref[...] = jax.lax.dot_general(
        hn, fW_ref[...], (((1,), (0,)), ((), ())),
        precision=_DF, preferred_element_type=jnp.float32) + fb_ref[...]


def _layer_mm(h2s, m, res, W, b, fW, fb):
    return pl.pallas_call(
        _layer_body,
        grid=(N // NB,),
        in_specs=[
            pl.BlockSpec((2, NB, DH), lambda i: (0, i, 0)),
            pl.BlockSpec((2, NB, DH), lambda i: (0, i, 0)),
            pl.BlockSpec((NB, D), lambda i: (i, 0)),
            pl.BlockSpec((D, D), lambda i: (0, 0)),
            pl.BlockSpec((1, D), lambda i: (0, 0)),
            pl.BlockSpec((D, DIMS), lambda i: (0, 0)),
            pl.BlockSpec((1, DIMS), lambda i: (0, 0)),
        ],
        out_specs=[
            pl.BlockSpec((NB, D), lambda i: (i, 0)),
            pl.BlockSpec((NB, DIMS), lambda i: (i, 0)),
        ],
        out_shape=[
            jax.ShapeDtypeStruct((N, D), jnp.float32),
            jax.ShapeDtypeStruct((N, DIMS), jnp.float32),
        ],
    )(h2s, m, res, W, b, fW, fb)


def _vnsum_body(h_ref, s_ref, b_ref, batch_ref, h2_ref, vnsum_ref):
    i = pl.program_id(0)

    @pl.when(i == 0)
    def _():
        vnsum_ref[...] = jnp.zeros_like(vnsum_ref)

    h2 = jnp.maximum(_ln(h_ref[...], s_ref[...], b_ref[...]), 0.0)
    h2_ref[...] = h2
    gids = lax.broadcasted_iota(jnp.int32, (G, NB), 0)
    oh = jnp.where(gids == batch_ref[0], 1.0, 0.0).astype(jnp.float32)
    vnsum_ref[...] += jax.lax.dot_general(
        oh, h2, (((1,), (0,)), ((), ())),
        precision=_HI, preferred_element_type=jnp.float32)


def _vnsum(h, s, b, batch3d):
    return pl.pallas_call(
        _vnsum_body,
        grid=(N // NB,),
        in_specs=[
            pl.BlockSpec((NB, D), lambda i: (i, 0)),
            pl.BlockSpec((1, D), lambda i: (0, 0)),
            pl.BlockSpec((1, D), lambda i: (0, 0)),
            pl.BlockSpec((1, 1, NB), lambda i: (i, 0, 0)),
        ],
        out_specs=[
            pl.BlockSpec((NB, D), lambda i: (i, 0)),
            pl.BlockSpec((G, D), lambda i: (0, 0)),
        ],
        out_shape=[
            jax.ShapeDtypeStruct((N, D), jnp.float32),
            jax.ShapeDtypeStruct((G, D), jnp.float32),
        ],
    )(h, s, b, batch3d)


def _vnmlp_body(vnsum_ref, vnprev_ref, W1_ref, b1_ref, s_ref, b_ref,
                W2_ref, b2_ref, out_ref):
    vt = vnsum_ref[...] + vnprev_ref[...]
    t = jax.lax.dot_general(vt, W1_ref[...], (((1,), (0,)), ((), ())),
                            precision=_DF,
                            preferred_element_type=jnp.float32) + b1_ref[...]
    t = jnp.maximum(_ln(t, s_ref[...], b_ref[...]), 0.0)
    out_ref[...] = jax.lax.dot_general(
        t, W2_ref[...], (((1,), (0,)), ((), ())),
        precision=_DF, preferred_element_type=jnp.float32) + b2_ref[...]


def _vnmlp(vnsum, vnprev, W1, b1, s, b, W2, b2):
    full = lambda a, bb: pl.BlockSpec((a, bb), lambda: (0, 0))
    return pl.pallas_call(
        _vnmlp_body,
        in_specs=[full(G, D), full(G, D), full(D, D), full(1, D),
                  full(1, D), full(1, D), full(D, D), full(1, D)],
        out_specs=full(G, D),
        out_shape=jax.ShapeDtypeStruct((G, D), jnp.float32),
    )(vnsum, vnprev, W1, b1, s, b, W2, b2)


def _vnadd_body(h2_ref, vn_ref, batch_ref, out_ref):
    bcol = batch_ref[0].reshape(NB, 1)
    gids = lax.broadcasted_iota(jnp.int32, (NB, G), 1)
    oh = jnp.where(bcol == gids, 1.0, 0.0).astype(jnp.float32)
    v = h2_ref[...] + jax.lax.dot_general(
        oh, vn_ref[...], (((1,), (0,)), ((), ())),
        precision=_HI, preferred_element_type=jnp.float32)
    out_ref[0] = v[:, :DH]
    out_ref[1] = v[:, DH:]


def _vnadd(h2, vn, batch3d):
    return pl.pallas_call(
        _vnadd_body,
        grid=(N // NB,),
        in_specs=[
            pl.BlockSpec((NB, D), lambda i: (i, 0)),
            pl.BlockSpec((G, D), lambda i: (0, 0)),
            pl.BlockSpec((1, 1, NB), lambda i: (i, 0, 0)),
        ],
        out_specs=pl.BlockSpec((2, NB, DH), lambda i: (0, i, 0)),
        out_shape=jax.ShapeDtypeStruct((2, N, DH), jnp.float32),
    )(h2, vn, batch3d)


def kernel(x, edge_attr, edge_index, batch, atom_emb, bond_emb, vn_table,
           gcn_W, gcn_b, ln_scale, ln_bias, ffn_W, ffn_b,
           vn_W1, vn_b1, vn_ln_s, vn_ln_b, vn_W2, vn_b2):
    dst = edge_index[1]
    attr_t = edge_attr.T.reshape(BOND_F, E)
    batch3d = batch.reshape(N // NB, 1, NB)
    zeros_pad = jnp.zeros((NPAD, DH), jnp.float32)
    zeros_nd = jnp.zeros((N, D), jnp.float32)

    h_init, h0s = _encode(x, atom_emb, vn_table)
    emb432 = _bond_table(bond_emb).reshape(432, DH)
    code2d, srcpre2d = _edge_codes(attr_t, edge_index)
    code = code2d.reshape(E)
    srcpre = srcpre2d.reshape(2 * E)
    vn = jnp.broadcast_to(vn_table, (G, D))

    outs = []

    h2s = h0s
    m = _agg_sc(h2s.reshape(2 * N, DH), srcpre, dst, code, emb432,
                zeros_pad).reshape(2, NPAD, DH)
    h, out0 = _layer_mm(h2s, m, zeros_nd, gcn_W[0],
                        gcn_b[0].reshape(1, D), ffn_W[0],
                        ffn_b[0].reshape(1, DIMS))
    outs.append(out0)

    for l in range(1, POWER):
        h2, vnsum = _vnsum(h, ln_scale[l - 1].reshape(1, D),
                           ln_bias[l - 1].reshape(1, D), batch3d)
        vn = _vnmlp(vnsum, vn, vn_W1[l - 1], vn_b1[l - 1].reshape(1, D),
                    vn_ln_s[l - 1].reshape(1, D), vn_ln_b[l - 1].reshape(1, D),
                    vn_W2[l - 1], vn_b2[l - 1].reshape(1, D))
        h2s = _vnadd(h2, vn, batch3d)
        m = _agg_sc(h2s.reshape(2 * N, DH), srcpre, dst, code, emb432,
                    zeros_pad).reshape(2, NPAD, DH)
        h, out_l = _layer_mm(h2s, m, h, gcn_W[l],
                             gcn_b[l].reshape(1, D), ffn_W[l],
                             ffn_b[l].reshape(1, DIMS))
        outs.append(out_l)

    h_graph = jnp.concatenate(outs, axis=-1)
    return (h_graph, h_init)
